# baseline (speedup 1.0000x reference)
import math
from fractions import Fraction

import numpy as np
import jax
import jax.numpy as jnp
from jax.experimental import pallas as pl

_L1, _L2 = 2, 2
_TLS = (0, 1, 2)


def _su2_cg(j1, m1, j2, m2, j3, m3):
    if m3 != m1 + m2:
        return 0.0
    vmin = int(max(-j1 + j2 + m3, -j1 + m1, 0))
    vmax = int(min(j2 + j3 + m1, j3 - j1 + j2, j3 + m3))
    f = math.factorial
    C = ((2.0 * j3 + 1.0) * Fraction(
        f(j3 + j1 - j2) * f(j3 - j1 + j2) * f(j1 + j2 - j3) * f(j3 + m3) * f(j3 - m3),
        f(j1 + j2 + j3 + 1) * f(j1 - m1) * f(j1 + m1) * f(j2 - m2) * f(j2 + m2))) ** 0.5
    S = 0
    for v in range(vmin, vmax + 1):
        S += (-1) ** (v + j2 + m2) * Fraction(
            f(j2 + j3 + m1 - v) * f(j1 - m1 + v),
            f(v) * f(j3 - j1 + j2 - v) * f(j3 + m3 - v) * f(v + j1 - j2 - m3))
    return float(C * S)


def _su2_clebsch_gordan(j1, j2, j3):
    mat = np.zeros((2 * j1 + 1, 2 * j2 + 1, 2 * j3 + 1), dtype=np.float64)
    if abs(j1 - j2) <= j3 <= j1 + j2:
        for m1 in range(-j1, j1 + 1):
            for m2 in range(-j2, j2 + 1):
                if abs(m1 + m2) <= j3:
                    mat[j1 + m1, j2 + m2, j3 + m1 + m2] = _su2_cg(j1, m1, j2, m2, j3, m1 + m2)
    return mat


def _change_basis_real_to_complex(l):
    q = np.zeros((2 * l + 1, 2 * l + 1), dtype=np.complex128)
    for m in range(-l, 0):
        q[l + m, l + abs(m)] = 1.0 / 2 ** 0.5
        q[l + m, l - abs(m)] = -1j / 2 ** 0.5
    q[l, l] = 1.0
    for m in range(1, l + 1):
        q[l + m, l + abs(m)] = (-1) ** m / 2 ** 0.5
        q[l + m, l - abs(m)] = 1j * (-1) ** m / 2 ** 0.5
    return (-1j) ** l * q


def _wigner_3j(l1, l2, l3):
    C = _su2_clebsch_gordan(l1, l2, l3).astype(np.complex128)
    Q1 = _change_basis_real_to_complex(l1)
    Q2 = _change_basis_real_to_complex(l2)
    Q3 = _change_basis_real_to_complex(l3)
    C = np.einsum('ij,kl,mn,ikn->jlm', Q1, Q2, np.conj(Q3.T), C)
    C = np.real(C)
    n = np.linalg.norm(C)
    if n > 0:
        C = C / n
    return C


def _build_w():
    ni, nj = 2 * _L1 + 1, 2 * _L2 + 1
    kt = sum(2 * l + 1 for l in _TLS)
    W = np.zeros((ni, nj, kt), dtype=np.float64)
    off = 0
    for l3 in range(abs(_L1 - _L2), _L1 + _L2 + 1):
        if l3 not in _TLS:
            continue
        cg = _wigner_3j(_L1, _L2, l3)
        mu1, mu2, mu3 = np.nonzero(cg)
        for i, j, k in zip(mu1, mu2, mu3):
            W[i, j, off + k] += cg[i, j, k]
        off += 2 * l3 + 1
    return W, ni, nj, kt


_Wnp, _NI, _NJ, _KT = _build_w()
_W1 = np.ascontiguousarray(
    np.transpose(_Wnp, (0, 2, 1)).reshape(_NI, _KT * _NJ)).astype(np.float32)
_T = np.zeros((_NJ, _KT * _NJ), dtype=np.float32)
for k in range(_KT):
    for j in range(_NJ):
        _T[j, k * _NJ + j] = 1.0
_S = np.zeros((_KT * _NJ, _KT), dtype=np.float32)
for k in range(_KT):
    for j in range(_NJ):
        _S[k * _NJ + j, k] = 1.0

_ROW_TILE = 4096


def _body(x_ref, y_ref, w1_ref, t_ref, s_ref, o_ref):
    t = jnp.dot(x_ref[...], w1_ref[...],
                preferred_element_type=jnp.float32,
                precision=jax.lax.Precision.HIGHEST)
    u = jnp.dot(y_ref[...], t_ref[...],
                preferred_element_type=jnp.float32,
                precision=jax.lax.Precision.HIGHEST)
    o_ref[...] = jnp.dot(t * u, s_ref[...],
                         preferred_element_type=jnp.float32,
                         precision=jax.lax.Precision.HIGHEST)


def kernel(x, y):
    B, F = x.shape[0], x.shape[1]
    N = B * F
    xf = x.reshape(N, _NI)
    yf = y.reshape(N, _NJ)
    w1 = jnp.asarray(_W1)
    tm = jnp.asarray(_T)
    sm = jnp.asarray(_S)
    grid = (pl.cdiv(N, _ROW_TILE),)
    out = pl.pallas_call(
        _body,
        grid=grid,
        in_specs=[
            pl.BlockSpec((_ROW_TILE, _NI), lambda i: (i, 0)),
            pl.BlockSpec((_ROW_TILE, _NJ), lambda i: (i, 0)),
            pl.BlockSpec((_NI, _KT * _NJ), lambda i: (0, 0)),
            pl.BlockSpec((_NJ, _KT * _NJ), lambda i: (0, 0)),
            pl.BlockSpec((_KT * _NJ, _KT), lambda i: (0, 0)),
        ],
        out_specs=pl.BlockSpec((_ROW_TILE, _KT), lambda i: (i, 0)),
        out_shape=jax.ShapeDtypeStruct((N, _KT), jnp.float32),
    )(xf, yf, w1, tm, sm)
    return out.reshape(B, F, _KT)


# speedup vs baseline: 6.3000x; 6.3000x over previous
import math
from fractions import Fraction

import numpy as np
import jax
import jax.numpy as jnp
from jax.experimental import pallas as pl

_L1, _L2 = 2, 2
_TLS = (0, 1, 2)


def _su2_cg(j1, m1, j2, m2, j3, m3):
    if m3 != m1 + m2:
        return 0.0
    vmin = int(max(-j1 + j2 + m3, -j1 + m1, 0))
    vmax = int(min(j2 + j3 + m1, j3 - j1 + j2, j3 + m3))
    f = math.factorial
    C = ((2.0 * j3 + 1.0) * Fraction(
        f(j3 + j1 - j2) * f(j3 - j1 + j2) * f(j1 + j2 - j3) * f(j3 + m3) * f(j3 - m3),
        f(j1 + j2 + j3 + 1) * f(j1 - m1) * f(j1 + m1) * f(j2 - m2) * f(j2 + m2))) ** 0.5
    S = 0
    for v in range(vmin, vmax + 1):
        S += (-1) ** (v + j2 + m2) * Fraction(
            f(j2 + j3 + m1 - v) * f(j1 - m1 + v),
            f(v) * f(j3 - j1 + j2 - v) * f(j3 + m3 - v) * f(v + j1 - j2 - m3))
    return float(C * S)


def _su2_clebsch_gordan(j1, j2, j3):
    mat = np.zeros((2 * j1 + 1, 2 * j2 + 1, 2 * j3 + 1), dtype=np.float64)
    if abs(j1 - j2) <= j3 <= j1 + j2:
        for m1 in range(-j1, j1 + 1):
            for m2 in range(-j2, j2 + 1):
                if abs(m1 + m2) <= j3:
                    mat[j1 + m1, j2 + m2, j3 + m1 + m2] = _su2_cg(j1, m1, j2, m2, j3, m1 + m2)
    return mat


def _change_basis_real_to_complex(l):
    q = np.zeros((2 * l + 1, 2 * l + 1), dtype=np.complex128)
    for m in range(-l, 0):
        q[l + m, l + abs(m)] = 1.0 / 2 ** 0.5
        q[l + m, l - abs(m)] = -1j / 2 ** 0.5
    q[l, l] = 1.0
    for m in range(1, l + 1):
        q[l + m, l + abs(m)] = (-1) ** m / 2 ** 0.5
        q[l + m, l - abs(m)] = 1j * (-1) ** m / 2 ** 0.5
    return (-1j) ** l * q


def _wigner_3j(l1, l2, l3):
    C = _su2_clebsch_gordan(l1, l2, l3).astype(np.complex128)
    Q1 = _change_basis_real_to_complex(l1)
    Q2 = _change_basis_real_to_complex(l2)
    Q3 = _change_basis_real_to_complex(l3)
    C = np.einsum('ij,kl,mn,ikn->jlm', Q1, Q2, np.conj(Q3.T), C)
    C = np.real(C)
    n = np.linalg.norm(C)
    if n > 0:
        C = C / n
    return C


def _build_w():
    ni, nj = 2 * _L1 + 1, 2 * _L2 + 1
    kt = sum(2 * l + 1 for l in _TLS)
    W = np.zeros((ni, nj, kt), dtype=np.float64)
    off = 0
    for l3 in range(abs(_L1 - _L2), _L1 + _L2 + 1):
        if l3 not in _TLS:
            continue
        cg = _wigner_3j(_L1, _L2, l3)
        mu1, mu2, mu3 = np.nonzero(cg)
        for i, j, k in zip(mu1, mu2, mu3):
            W[i, j, off + k] += cg[i, j, k]
        off += 2 * l3 + 1
    return W, ni, nj, kt


_Wnp, _NI, _NJ, _KT = _build_w()
_F = 16
_NP = _NI * _NJ

_A = np.zeros((_F * _NI, _F * _NP), dtype=np.float32)
_B = np.zeros((_F * _NJ, _F * _NP), dtype=np.float32)
_C = np.zeros((_F * _NP, _F * _KT), dtype=np.float32)
for f in range(_F):
    for i in range(_NI):
        for j in range(_NJ):
            c = f * _NP + i * _NJ + j
            _A[f * _NI + i, c] = 1.0
            _B[f * _NJ + j, c] = 1.0
            for k in range(_KT):
                if _Wnp[i, j, k] != 0.0:
                    _C[c, f * _KT + k] = _Wnp[i, j, k]

_ROW_TILE = 1600


def _body(x_ref, y_ref, a_ref, b_ref, c_ref, o_ref):
    xb = x_ref[...].astype(jnp.bfloat16)
    yb = y_ref[...].astype(jnp.bfloat16)
    xe = jnp.dot(xb, a_ref[...], preferred_element_type=jnp.float32)
    ye = jnp.dot(yb, b_ref[...], preferred_element_type=jnp.float32)
    p = (xe * ye).astype(jnp.bfloat16)
    o_ref[...] = jnp.dot(p, c_ref[...], preferred_element_type=jnp.float32)


def kernel(x, y):
    B, F = x.shape[0], x.shape[1]
    xf = x.reshape(B, F * _NI)
    yf = y.reshape(B, F * _NJ)
    am = jnp.asarray(_A, dtype=jnp.bfloat16)
    bm = jnp.asarray(_B, dtype=jnp.bfloat16)
    cm = jnp.asarray(_C, dtype=jnp.bfloat16)
    grid = (pl.cdiv(B, _ROW_TILE),)
    out = pl.pallas_call(
        _body,
        grid=grid,
        in_specs=[
            pl.BlockSpec((_ROW_TILE, F * _NI), lambda i: (i, 0)),
            pl.BlockSpec((_ROW_TILE, F * _NJ), lambda i: (i, 0)),
            pl.BlockSpec((_F * _NI, _F * _NP), lambda i: (0, 0)),
            pl.BlockSpec((_F * _NJ, _F * _NP), lambda i: (0, 0)),
            pl.BlockSpec((_F * _NP, _F * _KT), lambda i: (0, 0)),
        ],
        out_specs=pl.BlockSpec((_ROW_TILE, F * _KT), lambda i: (i, 0)),
        out_shape=jax.ShapeDtypeStruct((B, F * _KT), jnp.float32),
    )(xf, yf, am, bm, cm)
    return out.reshape(B, F, _KT)
